# Initial kernel scaffold; baseline (speedup 1.0000x reference)
import jax
import jax.numpy as jnp
from jax.experimental import pallas as pl


def kernel(sub, rel, edge_index, edge_type, ent2img, ent2desc, init_embed, init_rel, img_trans, desc_trans, c1_w_in, c1_w_out, c1_w_loop, c1_w_rel, c1_loop_rel, c1_bias, c2_w_in, c2_w_out, c2_w_loop, c2_w_rel, c2_loop_rel, c2_bias):
    raise NotImplementedError("write your pallas kernel here")


# speedup vs baseline: 10.6356x; 10.6356x over previous
import functools

import jax
import jax.numpy as jnp
from jax import lax
from jax.experimental import pallas as pl
from jax.experimental.pallas import tpu as pltpu
from jax.experimental.pallas import tpu_sc as plsc

NE = 10000
NP = 10240
D = 128
NT = 200
H = 160000
EP = 163840
EB = EP // 128
CT = NP * NT
NCORE, NSUB = 2, 16
C5 = NP // NCORE
CS = C5 * NT + 256
TB = EB // NSUB
ROWS_T = NP // NSUB

@functools.lru_cache(maxsize=1)
def _mesh():
    return plsc.VectorSubcoreMesh(
        core_axis_name="c", subcore_axis_name="s",
        num_cores=NCORE, num_subcores=NSUB)


def _base_body(img_ref, desc_ref, wi_ref, wd_ref, out_ref):
    out_ref[...] = (jnp.dot(img_ref[...], wi_ref[...],
                            preferred_element_type=jnp.float32)
                    + jnp.dot(desc_ref[...], wd_ref[...],
                              preferred_element_type=jnp.float32))


def _tc_base(ent2img, ent2desc, img_trans, desc_trans):
    g = 10
    rb = NE // g
    return pl.pallas_call(
        _base_body,
        grid=(g,),
        in_specs=[
            pl.BlockSpec((rb, 768), lambda i: (i, 0)),
            pl.BlockSpec((rb, 768), lambda i: (i, 0)),
            pl.BlockSpec((768, D), lambda i: (0, 0)),
            pl.BlockSpec((768, D), lambda i: (0, 0)),
        ],
        out_specs=pl.BlockSpec((rb, D), lambda i: (i, 0)),
        out_shape=jax.ShapeDtypeStruct((NE, D), jnp.float32),
    )(ent2img, ent2desc, img_trans, desc_trans)


def _rel_body(r0_ref, w1r_ref, w2r_ref, w1i_ref, w1o_ref, w2i_ref, w2o_ref,
              dst_ref, etl_ref,
              r1_ref, r2_ref, rw1_ref, rw2_ref, fidx_ref):
    r0 = r0_ref[...]
    r1 = jnp.dot(r0, w1r_ref[...], preferred_element_type=jnp.float32)
    r1_ref[...] = r1
    r2_ref[...] = jnp.dot(r1, w2r_ref[...], preferred_element_type=jnp.float32)
    rw1_ref[...] = jnp.stack([
        jnp.dot(r0[0:NT], w1i_ref[...], preferred_element_type=jnp.float32),
        jnp.dot(r0[NT:2 * NT], w1o_ref[...], preferred_element_type=jnp.float32)])
    rw2_ref[...] = jnp.stack([
        jnp.dot(r1[0:NT], w2i_ref[...], preferred_element_type=jnp.float32),
        jnp.dot(r1[NT:2 * NT], w2o_ref[...], preferred_element_type=jnp.float32)])
    dstv = dst_ref[...]
    etlv = etl_ref[...]
    dump = C5 * NT + (etlv & 7)
    percore = []
    for cc in range(NCORE):
        inr = (dstv >= cc * C5) & (dstv < (cc + 1) * C5) & (dstv < NE)
        percore.append(jnp.where(inr, (dstv - cc * C5) * NT + etlv, dump))
    fidx_ref[...] = jnp.stack(percore, axis=1)


def _tc_rel(init_rel, c1_w_rel, c2_w_rel, c1_w_in, c1_w_out, c2_w_in, c2_w_out,
            dst3, etl3):
    return pl.pallas_call(
        _rel_body,
        out_shape=(
            jax.ShapeDtypeStruct((2 * NT, D), jnp.float32),
            jax.ShapeDtypeStruct((2 * NT, D), jnp.float32),
            jax.ShapeDtypeStruct((2, NT, D), jnp.float32),
            jax.ShapeDtypeStruct((2, NT, D), jnp.float32),
            jax.ShapeDtypeStruct((2, NCORE, EB, 128), jnp.int32),
        ),
    )(init_rel, c1_w_rel, c2_w_rel, c1_w_in, c1_w_out, c2_w_in, c2_w_out,
      dst3, etl3)


def _dense_body(degi_ref, dego_ref, x_ref, base_ref, wi_ref, wo_ref, wl_ref,
                lr_ref, xs_ref, loop_ref, di_ref, do_ref):
    degi = degi_ref[...]
    dego = dego_ref[...]
    di = jnp.where(degi > 0, lax.rsqrt(degi), 0.0)
    do = jnp.where(dego > 0, lax.rsqrt(dego), 0.0)
    di_ref[...] = di
    do_ref[...] = do
    xm = x_ref[...] + base_ref[...]
    xs_ref[...] = jnp.stack([
        di * jnp.dot(xm, wi_ref[...], preferred_element_type=jnp.float32),
        do * jnp.dot(xm, wo_ref[...], preferred_element_type=jnp.float32)])
    loop_ref[...] = jnp.dot(xm - lr_ref[...], wl_ref[...],
                            preferred_element_type=jnp.float32)


def _tc_dense(degi, dego, x, base, w_in, w_out, w_loop, loop_rel):
    g = 10
    rb = NP // g
    return pl.pallas_call(
        _dense_body,
        grid=(g,),
        in_specs=[
            pl.BlockSpec((rb, 1), lambda i: (i, 0)),
            pl.BlockSpec((rb, 1), lambda i: (i, 0)),
            pl.BlockSpec((rb, D), lambda i: (i, 0)),
            pl.BlockSpec((rb, D), lambda i: (i, 0)),
            pl.BlockSpec((D, D), lambda i: (0, 0)),
            pl.BlockSpec((D, D), lambda i: (0, 0)),
            pl.BlockSpec((D, D), lambda i: (0, 0)),
            pl.BlockSpec((1, D), lambda i: (0, 0)),
        ],
        out_specs=(
            pl.BlockSpec((2, rb, D), lambda i: (0, i, 0)),
            pl.BlockSpec((rb, D), lambda i: (i, 0)),
            pl.BlockSpec((rb, 1), lambda i: (i, 0)),
            pl.BlockSpec((rb, 1), lambda i: (i, 0)),
        ),
        out_shape=(
            jax.ShapeDtypeStruct((2, NP, D), jnp.float32),
            jax.ShapeDtypeStruct((NP, D), jnp.float32),
            jax.ShapeDtypeStruct((NP, 1), jnp.float32),
            jax.ShapeDtypeStruct((NP, 1), jnp.float32),
        ),
    )(degi, dego, x, base, w_in, w_out, w_loop, loop_rel)


def _combine_body(p_ref, c_ref, rw_ref, di_ref, do_ref, loop_ref, bias_ref,
                  x_ref):
    p = p_ref[...]
    c = c_ref[...]
    rw = rw_ref[...]
    in_res = di_ref[...] * (p[0] - jnp.dot(c[0], rw[0],
                                           preferred_element_type=jnp.float32))
    out_res = do_ref[...] * (p[1] - jnp.dot(c[1], rw[1],
                                            preferred_element_type=jnp.float32))
    x_ref[...] = jnp.tanh((in_res + out_res + loop_ref[...]) * (1.0 / 3.0)
                          + bias_ref[...])


def _tc_combine(P, C, rw, di, do, loop, bias):
    g = 10
    rb = NP // g
    return pl.pallas_call(
        _combine_body,
        grid=(g,),
        in_specs=[
            pl.BlockSpec((2, rb, D), lambda i: (0, i, 0)),
            pl.BlockSpec((2, rb, NT), lambda i: (0, i, 0)),
            pl.BlockSpec((2, NT, D), lambda i: (0, 0, 0)),
            pl.BlockSpec((rb, 1), lambda i: (i, 0)),
            pl.BlockSpec((rb, 1), lambda i: (i, 0)),
            pl.BlockSpec((rb, D), lambda i: (i, 0)),
            pl.BlockSpec((1, D), lambda i: (0, 0)),
        ],
        out_specs=pl.BlockSpec((rb, D), lambda i: (i, 0)),
        out_shape=jax.ShapeDtypeStruct((NP, D), jnp.float32),
    )(P, C, rw, di, do, loop, bias)


def _combine_dense_body(p_ref, c_ref, rw_ref, di_ref, do_ref, loop_ref,
                        bias_ref, base_ref, wi_ref, wo_ref, wl_ref, lr_ref,
                        xs_ref, loop2_ref):
    p = p_ref[...]
    c = c_ref[...]
    rw = rw_ref[...]
    di = di_ref[...]
    do = do_ref[...]
    in_res = di * (p[0] - jnp.dot(c[0], rw[0],
                                  preferred_element_type=jnp.float32))
    out_res = do * (p[1] - jnp.dot(c[1], rw[1],
                                   preferred_element_type=jnp.float32))
    x2 = jnp.tanh((in_res + out_res + loop_ref[...]) * (1.0 / 3.0)
                  + bias_ref[...])
    xm2 = x2 + base_ref[...]
    xs_ref[...] = jnp.stack([
        di * jnp.dot(xm2, wi_ref[...], preferred_element_type=jnp.float32),
        do * jnp.dot(xm2, wo_ref[...], preferred_element_type=jnp.float32)])
    loop2_ref[...] = jnp.dot(xm2 - lr_ref[...], wl_ref[...],
                             preferred_element_type=jnp.float32)


def _tc_combine_dense(P, C, rw, di, do, loop, bias, base, w_in, w_out, w_loop,
                      loop_rel):
    g = 10
    rb = NP // g
    return pl.pallas_call(
        _combine_dense_body,
        grid=(g,),
        in_specs=[
            pl.BlockSpec((2, rb, D), lambda i: (0, i, 0)),
            pl.BlockSpec((2, rb, NT), lambda i: (0, i, 0)),
            pl.BlockSpec((2, NT, D), lambda i: (0, 0, 0)),
            pl.BlockSpec((rb, 1), lambda i: (i, 0)),
            pl.BlockSpec((rb, 1), lambda i: (i, 0)),
            pl.BlockSpec((rb, D), lambda i: (i, 0)),
            pl.BlockSpec((1, D), lambda i: (0, 0)),
            pl.BlockSpec((rb, D), lambda i: (i, 0)),
            pl.BlockSpec((D, D), lambda i: (0, 0)),
            pl.BlockSpec((D, D), lambda i: (0, 0)),
            pl.BlockSpec((D, D), lambda i: (0, 0)),
            pl.BlockSpec((1, D), lambda i: (0, 0)),
        ],
        out_specs=(
            pl.BlockSpec((2, rb, D), lambda i: (0, i, 0)),
            pl.BlockSpec((rb, D), lambda i: (i, 0)),
        ),
        out_shape=(
            jax.ShapeDtypeStruct((2, NP, D), jnp.float32),
            jax.ShapeDtypeStruct((NP, D), jnp.float32),
        ),
    )(P, C, rw, di, do, loop, bias, base, w_in, w_out, w_loop, loop_rel)


def _hist_body(src_hbm, zz_hbm, deg_hbm, idx_v, ones_v, zb_v, db_v, hist_sh):
    c = lax.axis_index("c")
    s = lax.axis_index("s")
    for k in range(8):
        ones_v[pl.ds(k * 16, 16)] = jnp.ones((16,), jnp.float32)
    pltpu.sync_copy(src_hbm.at[c, pl.ds(s * TB, TB)], idx_v)
    z = (2 * NP) // NSUB
    pltpu.sync_copy(zz_hbm.at[pl.ds(0, z)], zb_v)
    pltpu.sync_copy(zb_v, hist_sh.at[pl.ds(s * z, z)])
    plsc.subcore_barrier()

    def body(j, carry):
        pltpu.sync_copy(ones_v, hist_sh.at[idx_v.at[j]], add=True)
        return carry

    lax.fori_loop(0, TB, body, 0)
    plsc.subcore_barrier()
    pltpu.sync_copy(hist_sh.at[pl.ds(c * NP + s * ROWS_T, ROWS_T)], db_v)
    pltpu.sync_copy(db_v, deg_hbm.at[pl.ds(c * NP + s * ROWS_T, ROWS_T)])


def _sc_hist(src3, zz):
    return pl.kernel(
        _hist_body,
        out_type=jax.ShapeDtypeStruct((2 * NP,), jnp.float32),
        mesh=_mesh(),
        scratch_types=[
            pltpu.VMEM((TB, 128), jnp.int32),
            pltpu.VMEM((128,), jnp.float32),
            pltpu.VMEM(((2 * NP) // NSUB,), jnp.float32),
            pltpu.VMEM((ROWS_T,), jnp.float32),
            pltpu.VMEM_SHARED((2 * NP,), jnp.float32),
        ],
    )(src3, zz)


_ZCH = 8000


def _cmat_body(src_hbm, fidx_hbm, dinv_hbm, zz_hbm, cmat_hbm,
               sidx_v, fidx_v, vals_v, zb_v, db_v, c_sh, sem):
    c = lax.axis_index("c")
    s = lax.axis_index("s")
    dumpw = (C5 * NT) // NSUB
    pltpu.sync_copy(zz_hbm, zb_v)
    for h in range(2):
        pltpu.sync_copy(src_hbm.at[h, pl.ds(s * TB, TB)], sidx_v)
        pltpu.sync_copy(fidx_hbm.at[h, c, pl.ds(s * TB, TB)], fidx_v)
        for k in range(8):
            pltpu.sync_copy(zb_v,
                            c_sh.at[pl.ds(s * dumpw + k * _ZCH, _ZCH)])

        @pl.when(s == 0)
        def _zstrip():
            pltpu.sync_copy(zb_v.at[pl.ds(0, 256)],
                            c_sh.at[pl.ds(C5 * NT, 256)])

        plsc.subcore_barrier()

        def body(j, carry):
            pltpu.async_copy(dinv_hbm.at[sidx_v.at[j]], vals_v, sem).wait()
            pltpu.sync_copy(vals_v, c_sh.at[fidx_v.at[j]], add=True)
            return carry

        lax.fori_loop(0, TB, body, 0)
        plsc.subcore_barrier()
        for k in range(8):
            pltpu.sync_copy(c_sh.at[pl.ds(s * dumpw + k * _ZCH, _ZCH)], db_v)
            pltpu.sync_copy(
                db_v,
                cmat_hbm.at[pl.ds(h * CT + c * C5 * NT + s * dumpw + k * _ZCH,
                                  _ZCH)])
        plsc.subcore_barrier()


def _sc_cmat(src3, fidx4, dinv2f, zz):
    return pl.kernel(
        _cmat_body,
        out_type=jax.ShapeDtypeStruct((2 * CT,), jnp.float32),
        mesh=_mesh(),
        scratch_types=[
            pltpu.VMEM((TB, 128), jnp.int32),
            pltpu.VMEM((TB, 128), jnp.int32),
            pltpu.VMEM((128,), jnp.float32),
            pltpu.VMEM((_ZCH,), jnp.float32),
            pltpu.VMEM((_ZCH,), jnp.float32),
            pltpu.VMEM_SHARED((CS,), jnp.float32),
            pltpu.SemaphoreType.DMA,
        ],
    )(src3, fidx4, dinv2f, zz)


def _edges_body(src_hbm, dst_hbm, xs_hbm, zm_hbm, p_hbm,
                sidx_v, didx_v, rows_a, rows_b, acc_sh, sema, semb):
    c = lax.axis_index("c")
    s = lax.axis_index("s")
    pltpu.sync_copy(src_hbm.at[c, pl.ds(s * TB, TB)], sidx_v)
    pltpu.sync_copy(dst_hbm.at[c, pl.ds(s * TB, TB)], didx_v)
    pltpu.sync_copy(zm_hbm, rows_a)
    for k in range(ROWS_T // 128):
        pltpu.sync_copy(rows_a, acc_sh.at[pl.ds(s * ROWS_T + k * 128, 128)])
    plsc.subcore_barrier()

    def step(m, carry):
        pltpu.async_copy(xs_hbm.at[sidx_v.at[m]], rows_a, sema).wait()
        pltpu.sync_copy(rows_a, acc_sh.at[didx_v.at[m]], add=True)
        return carry

    lax.fori_loop(0, TB, step, 0)
    plsc.subcore_barrier()
    for k in range(ROWS_T // 128):
        pltpu.sync_copy(acc_sh.at[pl.ds(s * ROWS_T + k * 128, 128)], rows_a)
        pltpu.sync_copy(rows_a,
                        p_hbm.at[c, pl.ds(s * ROWS_T + k * 128, 128)])


def _sc_edges(src3, dst3, xs2f, zm):
    return pl.kernel(
        _edges_body,
        out_type=jax.ShapeDtypeStruct((2, NP, D), jnp.float32),
        mesh=_mesh(),
        scratch_types=[
            pltpu.VMEM((TB, 128), jnp.int32),
            pltpu.VMEM((TB, 128), jnp.int32),
            pltpu.VMEM((128, D), jnp.float32),
            pltpu.VMEM((128, D), jnp.float32),
            pltpu.VMEM_SHARED((NP, D), jnp.float32),
            pltpu.SemaphoreType.DMA,
            pltpu.SemaphoreType.DMA,
        ],
    )(src3, dst3, xs2f, zm)


def _gather_body(x_hbm, r_hbm, sub_hbm, rel_hbm, sube_hbm, rele_hbm,
                 sidx_v, ridx_v, srows_v, rrows_v, sem):
    c = lax.axis_index("c")
    s = lax.axis_index("s")
    wid = s * NCORE + c
    bw = 1024 // (NCORE * NSUB)
    pltpu.sync_copy(sub_hbm.at[pl.ds(wid * bw, bw)], sidx_v)
    pltpu.sync_copy(rel_hbm.at[pl.ds(wid * bw, bw)], ridx_v)
    pltpu.async_copy(x_hbm.at[sidx_v], srows_v, sem).wait()
    pltpu.sync_copy(srows_v, sube_hbm.at[pl.ds(wid * bw, bw)])
    pltpu.async_copy(r_hbm.at[ridx_v], rrows_v, sem).wait()
    pltpu.sync_copy(rrows_v, rele_hbm.at[pl.ds(wid * bw, bw)])


def _sc_gather(x_final, r_final, sub, rel):
    bw = 1024 // (NCORE * NSUB)
    return pl.kernel(
        _gather_body,
        out_type=(
            jax.ShapeDtypeStruct((1024, D), jnp.float32),
            jax.ShapeDtypeStruct((1024, D), jnp.float32),
        ),
        mesh=_mesh(),
        scratch_types=[
            pltpu.VMEM((bw,), jnp.int32),
            pltpu.VMEM((bw,), jnp.int32),
            pltpu.VMEM((bw, D), jnp.float32),
            pltpu.VMEM((bw, D), jnp.float32),
            pltpu.SemaphoreType.DMA,
        ],
    )(x_final, r_final, sub, rel)


def kernel(sub, rel, edge_index, edge_type, ent2img, ent2desc, init_embed,
           init_rel, img_trans, desc_trans,
           c1_w_in, c1_w_out, c1_w_loop, c1_w_rel, c1_loop_rel, c1_bias,
           c2_w_in, c2_w_out, c2_w_loop, c2_w_rel, c2_loop_rel, c2_bias):
    sub = sub.astype(jnp.int32)
    rel = rel.astype(jnp.int32)
    src = edge_index[0].astype(jnp.int32)
    dst = edge_index[1].astype(jnp.int32)
    et = edge_type.astype(jnp.int32)

    padn = EP - H
    padk = (jnp.arange(padn, dtype=jnp.int32) % 8) + NE
    zpad = jnp.zeros((padn,), jnp.int32)

    def padh(a, pv):
        return jnp.concatenate([a, pv])

    src3 = jnp.stack([padh(src[:H], padk),
                      padh(src[H:], padk) + NP]).reshape(2, EB, 128)
    dst3 = jnp.stack([padh(dst[:H], padk),
                      padh(dst[H:], padk)]).reshape(2, EB, 128)
    etl3 = jnp.stack([padh(et[:H], zpad),
                      padh(et[H:] - NT, zpad)]).reshape(2, EB, 128)

    zz = jnp.zeros((_ZCH,), jnp.float32)
    zm = jnp.zeros((128, D), jnp.float32)

    base = _tc_base(ent2img, ent2desc, img_trans, desc_trans)
    base_p = jnp.pad(base, ((0, NP - NE), (0, 0)))
    x1_p = jnp.pad(init_embed, ((0, NP - NE), (0, 0)))

    deg2 = _sc_hist(src3, zz).reshape(2, NP)
    degi = deg2[0].reshape(NP, 1)
    dego = deg2[1].reshape(NP, 1)

    r1, r2, rw1, rw2, fidx4 = _tc_rel(
        init_rel, c1_w_rel, c2_w_rel, c1_w_in, c1_w_out, c2_w_in, c2_w_out,
        dst3, etl3)

    xs1, loop1, di, do = _tc_dense(degi, dego, x1_p, base_p,
                                   c1_w_in, c1_w_out, c1_w_loop, c1_loop_rel)
    dinv2f = jnp.concatenate([di[:, 0], do[:, 0]])

    C2 = _sc_cmat(src3, fidx4, dinv2f, zz)
    C = C2.reshape(2, NP, NT)

    xs1f = xs1.reshape(2 * NP, D)
    P1 = _sc_edges(src3, dst3, xs1f, zm)

    b1 = c1_bias.reshape(1, D)
    b2 = c2_bias.reshape(1, D)
    xs2, loop2 = _tc_combine_dense(P1, C, rw1, di, do, loop1, b1, base_p,
                                   c2_w_in, c2_w_out, c2_w_loop, c2_loop_rel)

    xs2f = xs2.reshape(2 * NP, D)
    P2 = _sc_edges(src3, dst3, xs2f, zm)

    x_final = _tc_combine(P2, C, rw2, di, do, loop2, b2)

    sub_emb, rel_emb = _sc_gather(x_final, r2, sub, rel)
    return sub_emb, rel_emb, x_final[:NE]
